# baseline (speedup 1.0000x reference)
import jax
import jax.numpy as jnp
from jax.experimental import pallas as pl
from jax.experimental.pallas import tpu as pltpu

N = 256
FD = 128
ED = 64
TI = 32
NC = N // TI


def _edge_kernel(node_ref, adj_ref, wp1_ref, ba1_ref, wa2_ref, ba2_ref,
                 w1d_ref, b1_ref, g1_ref, be1_ref, w2_ref, b2_ref,
                 g2_ref, be2_ref, w3_ref, b3_ref, out_ref,
                 nf_s, nft_s, scores_s, rawt_s):
    s = pl.program_id(1)

    @pl.when(s == 0)
    def _precompute():
        x = node_ref[0]
        norm = jnp.sqrt(jnp.sum(x * x, axis=1, keepdims=True))
        nf = x / jnp.maximum(norm, 1e-12)
        nf_s[...] = nf
        nft_s[...] = nf.T

    @pl.when((s >= 1) & (s <= NC))
    def _edge_tile():
        t = s - 1
        nft = nft_s[...]

        def _sub(r0, c0, SI):
            MS = SI * N
            nfit = nf_s[pl.ds(r0, SI), :].T
            rawt_s[:FD, pl.ds(c0, MS)] = jnp.broadcast_to(
                nfit.reshape(FD, SI, 1), (FD, SI, N)).reshape(FD, MS)
            rawt_s[FD:, pl.ds(c0, MS)] = jnp.broadcast_to(
                nft.reshape(FD, 1, N), (FD, SI, N)).reshape(FD, MS)
            rawt = rawt_s[:, pl.ds(c0, MS)]

            p = jnp.dot(wp1_ref[...], rawt,
                        preferred_element_type=jnp.float32)
            a = jnp.maximum(p[:FD] + ba1_ref[...], 0.0)
            a = jnp.dot(wa2_ref[...], a,
                        preferred_element_type=jnp.float32) + ba2_ref[...]
            amax = jnp.max(a, axis=0, keepdims=True)
            ex = jnp.exp(a - amax)
            att = ex / jnp.sum(ex, axis=0, keepdims=True)

            datt = jnp.abs(rawt[:FD] - rawt[FD:]) * att
            h = (p[FD:] + jnp.dot(w1d_ref[...], datt,
                                  preferred_element_type=jnp.float32)
                 ) + b1_ref[...]
            h = (h / jnp.sqrt(jnp.float32(1.0 + 1e-5))) * g1_ref[...] + be1_ref[...]
            h = jnp.maximum(h, 0.0)
            h = jnp.dot(w2_ref[...], h,
                        preferred_element_type=jnp.float32) + b2_ref[...]
            h = (h / jnp.sqrt(jnp.float32(1.0 + 1e-5))) * g2_ref[...] + be2_ref[...]
            h = jnp.maximum(h, 0.0)
            logits = jnp.dot(w3_ref[...], h,
                             preferred_element_type=jnp.float32) + b3_ref[0, 0]
            scores_s[pl.ds(r0, SI), :] = jax.nn.sigmoid(logits.reshape(SI, N))

        half = TI // 2
        _sub(t * TI, 0, half)
        _sub(t * TI + half, half * N, half)

    @pl.when(s == NC + 1)
    def _select():
        scores = scores_s[...]
        mask = adj_ref[0] > 0.0
        ne = jnp.sum(mask.astype(jnp.int32))
        k = jnp.minimum(ne // 2, ne - 1)
        target = k + 1
        keys = jax.lax.bitcast_convert_type(scores, jnp.int32)
        keys = jnp.where(mask, keys, -1)

        def body(it, p):
            bit = 29 - it
            hi = p | jnp.left_shift(jnp.int32(1), bit)
            cnt = jnp.sum((keys >= hi).astype(jnp.int32))
            return jnp.where(cnt >= target, hi, p)

        p = jax.lax.fori_loop(0, 30, body, jnp.int32(0))
        out_ref[0] = jnp.where(keys >= p, scores, 0.0)


@jax.jit
def _run(node_feat, adj_matrix, wp1, ba1, wa2, ba2,
         w1d, b1, g1, be1, w2, b2, g2, be2, w3, b3):
    B = node_feat.shape[0]
    grid = (B, NC + 2)
    full = lambda b, s: (b, 0, 0)
    wspec = lambda shp: pl.BlockSpec(shp, lambda b, s: (0,) * len(shp))
    return pl.pallas_call(
        _edge_kernel,
        grid=grid,
        in_specs=[
            pl.BlockSpec((1, N, FD), full),
            pl.BlockSpec((1, N, N), full),
            wspec((FD + ED, 2 * FD)), wspec((FD, 1)),
            wspec((FD, FD)), wspec((FD, 1)),
            wspec((ED, FD)), wspec((ED, 1)), wspec((ED, 1)), wspec((ED, 1)),
            wspec((ED // 2, ED)), wspec((ED // 2, 1)),
            wspec((ED // 2, 1)), wspec((ED // 2, 1)),
            wspec((1, ED // 2)), wspec((1, 1)),
        ],
        out_specs=pl.BlockSpec((1, N, N), full),
        out_shape=jax.ShapeDtypeStruct((B, N, N), jnp.float32),
        scratch_shapes=[
            pltpu.VMEM((N, FD), jnp.float32),
            pltpu.VMEM((FD, N), jnp.float32),
            pltpu.VMEM((N, N), jnp.float32),
            pltpu.VMEM((2 * FD, TI * N), jnp.float32),
        ],
        compiler_params=pltpu.CompilerParams(
            dimension_semantics=("arbitrary", "arbitrary"),
        ),
    )(node_feat, adj_matrix, wp1, ba1, wa2, ba2,
      w1d, b1, g1, be1, w2, b2, g2, be2, w3, b3)


def kernel(node_feat, adj_matrix, Wa1, ba1, Wa2, ba2, W1, b1, g1, be1,
           W2, b2, g2, be2, W3, b3, current_epoch, warmup_epochs,
           temperature, graph_size_adaptation, min_edges_per_node):
    wp1 = jnp.concatenate([Wa1, W1[:, :2 * FD]], axis=0)
    return _run(
        node_feat, adj_matrix,
        wp1, ba1.reshape(FD, 1),
        Wa2, ba2.reshape(FD, 1),
        W1[:, 2 * FD:], b1.reshape(ED, 1), g1.reshape(ED, 1),
        be1.reshape(ED, 1),
        W2, b2.reshape(ED // 2, 1), g2.reshape(ED // 2, 1),
        be2.reshape(ED // 2, 1),
        W3, b3.reshape(1, 1),
    )


# speedup vs baseline: 1.1449x; 1.1449x over previous
import jax
import jax.numpy as jnp
from jax.experimental import pallas as pl
from jax.experimental.pallas import tpu as pltpu

N = 256
FD = 128
ED = 64
TI = 64
NC = N // TI


def _edge_kernel(node_ref, adj_ref, wp1_ref, ba1_ref, wa2_ref, ba2_ref,
                 w1d_ref, b1_ref, g1_ref, be1_ref, w2_ref, b2_ref,
                 g2_ref, be2_ref, w3_ref, b3_ref, out_ref,
                 nf_s, nft_s, scores_s, rawt_s):
    s = pl.program_id(1)

    @pl.when(s == 0)
    def _precompute():
        x = node_ref[0]
        norm = jnp.sqrt(jnp.sum(x * x, axis=1, keepdims=True))
        nf = x / jnp.maximum(norm, 1e-12)
        nf_s[...] = nf
        nft_s[...] = nf.T

    @pl.when((s >= 1) & (s <= NC))
    def _edge_tile():
        t = s - 1
        r0 = t * TI
        M = TI * N
        nft = nft_s[...]
        nfit = nf_s[pl.ds(r0, TI), :].T
        rawt_s[:FD, :] = jnp.broadcast_to(nfit.reshape(FD, TI, 1),
                                          (FD, TI, N)).reshape(FD, M)
        rawt_s[FD:, :] = jnp.broadcast_to(nft.reshape(FD, 1, N),
                                          (FD, TI, N)).reshape(FD, M)
        rawt = rawt_s[...]

        p = jnp.dot(wp1_ref[...], rawt,
                    preferred_element_type=jnp.float32)
        a = jnp.maximum(p[:FD] + ba1_ref[...], 0.0)
        a = jnp.dot(wa2_ref[...], a,
                    preferred_element_type=jnp.float32) + ba2_ref[...]
        amax = jnp.max(a, axis=0, keepdims=True)
        ex = jnp.exp(a - amax)
        att = ex / jnp.sum(ex, axis=0, keepdims=True)

        datt = jnp.abs(rawt[:FD] - rawt[FD:]) * att
        h = (p[FD:] + jnp.dot(w1d_ref[...], datt,
                              preferred_element_type=jnp.float32)) + b1_ref[...]
        h = (h / jnp.sqrt(jnp.float32(1.0 + 1e-5))) * g1_ref[...] + be1_ref[...]
        h = jnp.maximum(h, 0.0)
        h = jnp.dot(w2_ref[...], h,
                    preferred_element_type=jnp.float32) + b2_ref[...]
        h = (h / jnp.sqrt(jnp.float32(1.0 + 1e-5))) * g2_ref[...] + be2_ref[...]
        h = jnp.maximum(h, 0.0)
        logits = jnp.dot(w3_ref[...], h,
                         preferred_element_type=jnp.float32) + b3_ref[0, 0]
        scores_s[pl.ds(r0, TI), :] = jax.nn.sigmoid(logits.reshape(TI, N))

    @pl.when(s == NC + 1)
    def _select():
        scores = scores_s[...]
        mask = adj_ref[0] > 0.0
        ne = jnp.sum(mask.astype(jnp.int32))
        k = jnp.minimum(ne // 2, ne - 1)
        target = k + 1
        keys = jax.lax.bitcast_convert_type(scores, jnp.int32)
        keys = jnp.where(mask, keys, -1)

        def body(it, p):
            bit = 29 - it
            hi = p | jnp.left_shift(jnp.int32(1), bit)
            cnt = jnp.sum((keys >= hi).astype(jnp.int32))
            return jnp.where(cnt >= target, hi, p)

        p = jax.lax.fori_loop(0, 30, body, jnp.int32(0))
        out_ref[0] = jnp.where(keys >= p, scores, 0.0)


@jax.jit
def _run(node_feat, adj_matrix, wp1, ba1, wa2, ba2,
         w1d, b1, g1, be1, w2, b2, g2, be2, w3, b3):
    B = node_feat.shape[0]
    grid = (B, NC + 2)
    full = lambda b, s: (b, 0, 0)
    wspec = lambda shp: pl.BlockSpec(shp, lambda b, s: (0,) * len(shp))
    return pl.pallas_call(
        _edge_kernel,
        grid=grid,
        in_specs=[
            pl.BlockSpec((1, N, FD), full),
            pl.BlockSpec((1, N, N), full),
            wspec((FD + ED, 2 * FD)), wspec((FD, 1)),
            wspec((FD, FD)), wspec((FD, 1)),
            wspec((ED, FD)), wspec((ED, 1)), wspec((ED, 1)), wspec((ED, 1)),
            wspec((ED // 2, ED)), wspec((ED // 2, 1)),
            wspec((ED // 2, 1)), wspec((ED // 2, 1)),
            wspec((1, ED // 2)), wspec((1, 1)),
        ],
        out_specs=pl.BlockSpec((1, N, N), full),
        out_shape=jax.ShapeDtypeStruct((B, N, N), jnp.float32),
        scratch_shapes=[
            pltpu.VMEM((N, FD), jnp.float32),
            pltpu.VMEM((FD, N), jnp.float32),
            pltpu.VMEM((N, N), jnp.float32),
            pltpu.VMEM((2 * FD, TI * N), jnp.float32),
        ],
        compiler_params=pltpu.CompilerParams(
            dimension_semantics=("arbitrary", "arbitrary"),
        ),
    )(node_feat, adj_matrix, wp1, ba1, wa2, ba2,
      w1d, b1, g1, be1, w2, b2, g2, be2, w3, b3)


def kernel(node_feat, adj_matrix, Wa1, ba1, Wa2, ba2, W1, b1, g1, be1,
           W2, b2, g2, be2, W3, b3, current_epoch, warmup_epochs,
           temperature, graph_size_adaptation, min_edges_per_node):
    wp1 = jnp.concatenate([Wa1, W1[:, :2 * FD]], axis=0)
    return _run(
        node_feat, adj_matrix,
        wp1, ba1.reshape(FD, 1),
        Wa2, ba2.reshape(FD, 1),
        W1[:, 2 * FD:], b1.reshape(ED, 1), g1.reshape(ED, 1),
        be1.reshape(ED, 1),
        W2, b2.reshape(ED // 2, 1), g2.reshape(ED // 2, 1),
        be2.reshape(ED // 2, 1),
        W3, b3.reshape(1, 1),
    )


# speedup vs baseline: 1.1934x; 1.0424x over previous
import jax
import jax.numpy as jnp
from jax.experimental import pallas as pl
from jax.experimental.pallas import tpu as pltpu

N = 256
FD = 128
ED = 64
TI = 128
NC = N // TI


def _edge_kernel(node_ref, adj_ref, wp1_ref, ba1_ref, wa2_ref, ba2_ref,
                 w1d_ref, b1_ref, g1_ref, be1_ref, w2_ref, b2_ref,
                 g2_ref, be2_ref, w3_ref, b3_ref, out_ref,
                 nf_s, nft_s, scores_s, rawt_s):
    s = pl.program_id(1)

    @pl.when(s == 0)
    def _precompute():
        x = node_ref[0]
        norm = jnp.sqrt(jnp.sum(x * x, axis=1, keepdims=True))
        nf = x / jnp.maximum(norm, 1e-12)
        nf_s[...] = nf
        nft_s[...] = nf.T

    @pl.when((s >= 1) & (s <= NC))
    def _edge_tile():
        t = s - 1
        r0 = t * TI
        M = TI * N
        nft = nft_s[...]
        nfit = nf_s[pl.ds(r0, TI), :].T
        rawt_s[:FD, :] = jnp.broadcast_to(nfit.reshape(FD, TI, 1),
                                          (FD, TI, N)).reshape(FD, M)
        rawt_s[FD:, :] = jnp.broadcast_to(nft.reshape(FD, 1, N),
                                          (FD, TI, N)).reshape(FD, M)
        rawt = rawt_s[...]

        p = jnp.dot(wp1_ref[...], rawt,
                    preferred_element_type=jnp.float32)
        a = jnp.maximum(p[:FD] + ba1_ref[...], 0.0)
        a = jnp.dot(wa2_ref[...], a,
                    preferred_element_type=jnp.float32) + ba2_ref[...]
        amax = jnp.max(a, axis=0, keepdims=True)
        ex = jnp.exp(a - amax)
        att = ex / jnp.sum(ex, axis=0, keepdims=True)

        datt = jnp.abs(rawt[:FD] - rawt[FD:]) * att
        h = (p[FD:] + jnp.dot(w1d_ref[...], datt,
                              preferred_element_type=jnp.float32)) + b1_ref[...]
        h = (h / jnp.sqrt(jnp.float32(1.0 + 1e-5))) * g1_ref[...] + be1_ref[...]
        h = jnp.maximum(h, 0.0)
        h = jnp.dot(w2_ref[...], h,
                    preferred_element_type=jnp.float32) + b2_ref[...]
        h = (h / jnp.sqrt(jnp.float32(1.0 + 1e-5))) * g2_ref[...] + be2_ref[...]
        h = jnp.maximum(h, 0.0)
        logits = jnp.dot(w3_ref[...], h,
                         preferred_element_type=jnp.float32) + b3_ref[0, 0]
        scores_s[pl.ds(r0, TI), :] = jax.nn.sigmoid(logits.reshape(TI, N))

    @pl.when(s == NC + 1)
    def _select():
        scores = scores_s[...]
        mask = adj_ref[0] > 0.0
        ne = jnp.sum(mask.astype(jnp.int32))
        k = jnp.minimum(ne // 2, ne - 1)
        target = k + 1
        keys = jax.lax.bitcast_convert_type(scores, jnp.int32)
        keys = jnp.where(mask, keys, -1)

        def body(it, p):
            bit = 29 - it
            hi = p | jnp.left_shift(jnp.int32(1), bit)
            cnt = jnp.sum((keys >= hi).astype(jnp.int32))
            return jnp.where(cnt >= target, hi, p)

        p = jax.lax.fori_loop(0, 30, body, jnp.int32(0))
        out_ref[0] = jnp.where(keys >= p, scores, 0.0)


@jax.jit
def _run(node_feat, adj_matrix, wp1, ba1, wa2, ba2,
         w1d, b1, g1, be1, w2, b2, g2, be2, w3, b3):
    B = node_feat.shape[0]
    grid = (B, NC + 2)
    full = lambda b, s: (b, 0, 0)
    wspec = lambda shp: pl.BlockSpec(shp, lambda b, s: (0,) * len(shp))
    return pl.pallas_call(
        _edge_kernel,
        grid=grid,
        in_specs=[
            pl.BlockSpec((1, N, FD), full),
            pl.BlockSpec((1, N, N), full),
            wspec((FD + ED, 2 * FD)), wspec((FD, 1)),
            wspec((FD, FD)), wspec((FD, 1)),
            wspec((ED, FD)), wspec((ED, 1)), wspec((ED, 1)), wspec((ED, 1)),
            wspec((ED // 2, ED)), wspec((ED // 2, 1)),
            wspec((ED // 2, 1)), wspec((ED // 2, 1)),
            wspec((1, ED // 2)), wspec((1, 1)),
        ],
        out_specs=pl.BlockSpec((1, N, N), full),
        out_shape=jax.ShapeDtypeStruct((B, N, N), jnp.float32),
        scratch_shapes=[
            pltpu.VMEM((N, FD), jnp.float32),
            pltpu.VMEM((FD, N), jnp.float32),
            pltpu.VMEM((N, N), jnp.float32),
            pltpu.VMEM((2 * FD, TI * N), jnp.float32),
        ],
        compiler_params=pltpu.CompilerParams(
            dimension_semantics=("arbitrary", "arbitrary"),
        ),
    )(node_feat, adj_matrix, wp1, ba1, wa2, ba2,
      w1d, b1, g1, be1, w2, b2, g2, be2, w3, b3)


def kernel(node_feat, adj_matrix, Wa1, ba1, Wa2, ba2, W1, b1, g1, be1,
           W2, b2, g2, be2, W3, b3, current_epoch, warmup_epochs,
           temperature, graph_size_adaptation, min_edges_per_node):
    wp1 = jnp.concatenate([Wa1, W1[:, :2 * FD]], axis=0)
    return _run(
        node_feat, adj_matrix,
        wp1, ba1.reshape(FD, 1),
        Wa2, ba2.reshape(FD, 1),
        W1[:, 2 * FD:], b1.reshape(ED, 1), g1.reshape(ED, 1),
        be1.reshape(ED, 1),
        W2, b2.reshape(ED // 2, 1), g2.reshape(ED // 2, 1),
        be2.reshape(ED // 2, 1),
        W3, b3.reshape(1, 1),
    )
